# baseline (speedup 1.0000x reference)
import jax
import jax.numpy as jnp
from jax.experimental import pallas as pl
from jax.experimental.pallas import tpu as pltpu

N = 4096
D = 512
H = 128
T2 = 512
NT2 = N // T2
THR = 0.9
SUS_THR = 0.88
_BN_SCALE = 1.0 / (1.0 + 1e-5) ** 0.5


def _gcn_kernel(x_ref,
                wc0, wc1, wc2, l1w, l1b, bg, bb, l2w, l2b,
                out_ref,
                v1a_s, v1b_s, v2_s, base_s, h1_s, h2_s, h3_s, dinv_s, acc_s,
                nrm_s, xnb_s, cnt_sm):
    nrm_ref = nrm_s

    def norm_body(i, c):
        rows = pl.ds(i * T2, T2)
        xi = x_ref[rows, :]
        nrm = jnp.maximum(jnp.sqrt(jnp.sum(xi * xi, axis=1, keepdims=True)),
                          1e-12)
        nrm_s[rows, :] = nrm
        xnb_s[rows, :] = (xi / nrm).astype(jnp.bfloat16)
        return c

    jax.lax.fori_loop(0, NT2, norm_body, 0)

    def zero_body(k, c):
        cnt_sm[k] = 0.0
        return c

    jax.lax.fori_loop(0, NT2 * NT2, zero_body, 0)

    def pair_body(k, c):
        i = k // NT2
        j = k % NT2

        @pl.when(j >= i)
        def _():
            xib = xnb_s[pl.ds(i * T2, T2), :]
            xjb = xnb_s[pl.ds(j * T2, T2), :]
            s = jax.lax.dot_general(xib, xjb, (((1,), (1,)), ((), ())),
                                    preferred_element_type=jnp.float32)

            @pl.when(j > i)
            def _():
                v = jnp.sum((s > SUS_THR).astype(jnp.float32))
                cnt_sm[i * NT2 + j] += v
                cnt_sm[j * NT2 + i] += v

            @pl.when(j == i)
            def _():
                colg = jax.lax.broadcasted_iota(jnp.int32, (T2, T2), 1)
                rowg = jax.lax.broadcasted_iota(jnp.int32, (T2, T2), 0)
                sus = ((s > SUS_THR) & (colg > rowg)).astype(jnp.float32)
                cnt_sm[i * NT2 + i] += 2.0 * jnp.sum(sus)
        return c

    jax.lax.fori_loop(0, NT2 * NT2, pair_body, 0)

    def dv_of(i):
        return dinv_s[pl.ds(i * T2, T2), :]

    def row_active(i):
        def rbody(j, r):
            return jnp.maximum(r, cnt_sm[i * NT2 + j])
        return jax.lax.fori_loop(0, NT2, rbody, jnp.float32(0.0))

    def exact_tile(i, j):
        rows = pl.ds(i * T2, T2)
        cols = pl.ds(j * T2, T2)
        xi = x_ref[rows, :] / nrm_ref[rows, :]
        xj = x_ref[cols, :] / nrm_ref[cols, :]
        s = jax.lax.dot_general(xi, xj, (((1,), (1,)), ((), ())),
                                preferred_element_type=jnp.float32)
        colg = jax.lax.broadcasted_iota(jnp.int32, (T2, T2), 1) + j * T2
        rowg = jax.lax.broadcasted_iota(jnp.int32, (T2, T2), 0) + i * T2
        return jnp.where((s > THR) & (colg != rowg), s, 0.0)

    def deg_dinv(i):
        rows = pl.ds(i * T2, T2)
        act = row_active(i)

        @pl.when(act > 0.5)
        def _():
            acc_s[...] = jnp.zeros((T2, H), jnp.float32)

            def dbody(j, carry):
                @pl.when(cnt_sm[i * NT2 + j] > 0.5)
                def _():
                    a = exact_tile(i, j)
                    acc_s[:, 0:1] += jnp.sum(a, axis=1, keepdims=True)
                return carry

            jax.lax.fori_loop(0, NT2, dbody, 0)
            deg = acc_s[:, 0:1]
            dinv_s[rows, :] = jnp.where(
                deg > 0, jax.lax.rsqrt(jnp.maximum(deg, 1e-12)), 0.0)

        @pl.when(act <= 0.5)
        def _():
            dinv_s[rows, :] = jnp.zeros((T2, 1), jnp.float32)

    def phase_a(i, h, wcat, v1_s):
        rows = pl.ds(i * T2, T2)
        bvv = jnp.dot(h, wcat[...], preferred_element_type=jnp.float32)
        b0, v1, v2 = bvv[:, :H], bvv[:, H:2 * H], bvv[:, 2 * H:]
        base_s[rows, :] = b0 - v2

        @pl.when(row_active(i) > 0.5)
        def _():
            v1_s[rows, :] = v1
            v2_s[rows, :] = dv_of(i) * v2

    def spmm_rows(i, src_s):
        acc_s[...] = jnp.zeros((T2, H), jnp.float32)

        def body(j, carry):
            @pl.when(cnt_sm[i * NT2 + j] > 0.5)
            def _():
                a = exact_tile(i, j)
                acc_s[...] += jnp.dot(a, src_s[pl.ds(j * T2, T2), :],
                                      preferred_element_type=jnp.float32)
            return carry

        jax.lax.fori_loop(0, NT2, body, 0)
        return -dv_of(i) * acc_s[...]

    def phase_b(i, v1_s):
        rows = pl.ds(i * T2, T2)
        act = row_active(i)

        @pl.when(act > 0.5)
        def _():
            m2 = spmm_rows(i, v2_s)
            v1_s[rows, :] = dv_of(i) * (v1_s[rows, :] + 2.0 * m2)

    def relu_into(i, v1_s, dst_s):
        rows = pl.ds(i * T2, T2)
        act = row_active(i)

        @pl.when(act > 0.5)
        def _():
            dst_s[rows, :] = jnp.maximum(
                base_s[rows, :] + spmm_rows(i, v1_s), 0.0)

        @pl.when(act <= 0.5)
        def _():
            dst_s[rows, :] = jnp.maximum(base_s[rows, :], 0.0)

    def head(i):
        rows = pl.ds(i * T2, T2)
        z = (jnp.dot(h1_s[rows, :], l1w[:H, :],
                     preferred_element_type=jnp.float32)
             + jnp.dot(h2_s[rows, :], l1w[H:2 * H, :],
                       preferred_element_type=jnp.float32)
             + jnp.dot(h3_s[rows, :], l1w[2 * H:, :],
                       preferred_element_type=jnp.float32)
             + l1b[...])
        z = jnp.maximum(z, 0.0)
        z = z * (_BN_SCALE * bg[...]) + bb[...]
        logit = (jnp.dot(z, l2w[...], preferred_element_type=jnp.float32)
                 + l2b[...])
        m = jnp.max(logit, axis=1, keepdims=True)
        e = jnp.exp(logit - m)
        out_ref[rows, :] = e / jnp.sum(e, axis=1, keepdims=True)

    def loop(fn):
        jax.lax.fori_loop(0, NT2, lambda i, c: (fn(i), c)[1], 0)

    def dega0(i):
        deg_dinv(i)
        phase_a(i, x_ref[pl.ds(i * T2, T2), :], wc0, v1a_s)

    loop(dega0)
    loop(lambda i: phase_b(i, v1a_s))

    def c0a1(i):
        relu_into(i, v1a_s, h1_s)
        phase_a(i, h1_s[pl.ds(i * T2, T2), :], wc1, v1b_s)

    loop(c0a1)
    loop(lambda i: phase_b(i, v1b_s))

    def c1a2(i):
        relu_into(i, v1b_s, h2_s)
        phase_a(i, h2_s[pl.ds(i * T2, T2), :], wc2, v1a_s)

    loop(c1a2)
    loop(lambda i: phase_b(i, v1a_s))

    def c2head(i):
        relu_into(i, v1a_s, h3_s)
        head(i)

    loop(c2head)


def kernel(x, w0_0, w0_1, w0_2, w1_0, w1_1, w1_2, w2_0, w2_1, w2_2,
           lin1_w, lin1_b, bn_gamma, bn_beta, lin2_w, lin2_b):
    f32 = jnp.float32
    full = pl.BlockSpec(memory_space=pltpu.VMEM)
    wc0 = jnp.concatenate([w0_0, w0_1, w0_2], axis=1)
    wc1 = jnp.concatenate([w1_0, w1_1, w1_2], axis=1)
    wc2 = jnp.concatenate([w2_0, w2_1, w2_2], axis=1)
    out = pl.pallas_call(
        _gcn_kernel,
        in_specs=[full] * 10,
        out_specs=full,
        out_shape=jax.ShapeDtypeStruct((N, 10), f32),
        scratch_shapes=[pltpu.VMEM((N, H), f32)] * 7
                       + [pltpu.VMEM((N, 1), f32)]
                       + [pltpu.VMEM((T2, H), f32)]
                       + [pltpu.VMEM((N, 1), f32),
                          pltpu.VMEM((N, D), jnp.bfloat16),
                          pltpu.SMEM((NT2 * NT2,), f32)],
    )(x, wc0, wc1, wc2,
      lin1_w, lin1_b.reshape(1, -1), bn_gamma.reshape(1, -1),
      bn_beta.reshape(1, -1), lin2_w, lin2_b.reshape(1, -1))
    return out


# speedup vs baseline: 1.0907x; 1.0907x over previous
import jax
import jax.numpy as jnp
from jax.experimental import pallas as pl
from jax.experimental.pallas import tpu as pltpu

N = 4096
D = 512
H = 128
T2 = 512
NT2 = N // T2
THR = 0.9
SUS_THR = 0.88
_BN_SCALE = 1.0 / (1.0 + 1e-5) ** 0.5


def _gcn_kernel(x_ref,
                wc0, wc1, wc2, l1w, l1b, bg, bb, l2w, l2b,
                out_ref,
                v1a_s, v1b_s, v2_s, base_s, h1_s, h2_s, h3_s, dinv_s, acc_s,
                nrm_s, xnb_s, cnt_sm):
    nrm_ref = nrm_s

    def norm_body(i, c):
        rows = pl.ds(i * T2, T2)
        xi = x_ref[rows, :]
        nrm = jnp.maximum(jnp.sqrt(jnp.sum(xi * xi, axis=1, keepdims=True)),
                          1e-12)
        nrm_s[rows, :] = nrm
        xnb_s[rows, :] = (xi / nrm).astype(jnp.bfloat16)
        return c

    jax.lax.fori_loop(0, NT2, norm_body, 0)

    def zero_body(k, c):
        cnt_sm[k] = 0.0
        return c

    jax.lax.fori_loop(0, NT2 * NT2, zero_body, 0)

    def pair_body(k, c):
        i = k // NT2
        j = k % NT2

        @pl.when(j >= i)
        def _():
            xib = xnb_s[pl.ds(i * T2, T2), :]
            xjb = xnb_s[pl.ds(j * T2, T2), :]
            s = jax.lax.dot_general(xib, xjb, (((1,), (1,)), ((), ())),
                                    preferred_element_type=jnp.float32)
            colg = jax.lax.broadcasted_iota(jnp.int32, (T2, T2), 1)
            rowg = jax.lax.broadcasted_iota(jnp.int32, (T2, T2), 0)
            off_diag = (colg > rowg) | (j != i)
            sus = ((s > SUS_THR) & off_diag).astype(jnp.float32)
            v = jnp.sum(sus)
            cnt_sm[i * NT2 + j] += v
            cnt_sm[j * NT2 + i] += v
        return c

    jax.lax.fori_loop(0, NT2 * NT2, pair_body, 0)

    def dv_of(i):
        return dinv_s[pl.ds(i * T2, T2), :]

    def row_active(i):
        def rbody(j, r):
            return jnp.maximum(r, cnt_sm[i * NT2 + j])
        return jax.lax.fori_loop(0, NT2, rbody, jnp.float32(0.0))

    def exact_tile(i, j):
        rows = pl.ds(i * T2, T2)
        cols = pl.ds(j * T2, T2)
        xi = x_ref[rows, :] / nrm_ref[rows, :]
        xj = x_ref[cols, :] / nrm_ref[cols, :]
        s = jax.lax.dot_general(xi, xj, (((1,), (1,)), ((), ())),
                                preferred_element_type=jnp.float32)
        colg = jax.lax.broadcasted_iota(jnp.int32, (T2, T2), 1) + j * T2
        rowg = jax.lax.broadcasted_iota(jnp.int32, (T2, T2), 0) + i * T2
        return jnp.where((s > THR) & (colg != rowg), s, 0.0)

    def deg_dinv(i):
        rows = pl.ds(i * T2, T2)
        act = row_active(i)

        @pl.when(act > 0.5)
        def _():
            acc_s[...] = jnp.zeros((T2, H), jnp.float32)

            def dbody(j, carry):
                @pl.when(cnt_sm[i * NT2 + j] > 0.5)
                def _():
                    a = exact_tile(i, j)
                    acc_s[:, 0:1] += jnp.sum(a, axis=1, keepdims=True)
                return carry

            jax.lax.fori_loop(0, NT2, dbody, 0)
            deg = acc_s[:, 0:1]
            dinv_s[rows, :] = jnp.where(
                deg > 0, jax.lax.rsqrt(jnp.maximum(deg, 1e-12)), 0.0)

        @pl.when(act <= 0.5)
        def _():
            dinv_s[rows, :] = jnp.zeros((T2, 1), jnp.float32)

    def phase_a(i, h, wcat, v1_s):
        rows = pl.ds(i * T2, T2)
        bvv = jnp.dot(h, wcat[...], preferred_element_type=jnp.float32)
        b0, v1, v2 = bvv[:, :H], bvv[:, H:2 * H], bvv[:, 2 * H:]
        base_s[rows, :] = b0 - v2

        @pl.when(row_active(i) > 0.5)
        def _():
            v1_s[rows, :] = v1
            v2_s[rows, :] = dv_of(i) * v2

    def spmm_rows(i, src_s):
        acc_s[...] = jnp.zeros((T2, H), jnp.float32)

        def body(j, carry):
            @pl.when(cnt_sm[i * NT2 + j] > 0.5)
            def _():
                a = exact_tile(i, j)
                acc_s[...] += jnp.dot(a, src_s[pl.ds(j * T2, T2), :],
                                      preferred_element_type=jnp.float32)
            return carry

        jax.lax.fori_loop(0, NT2, body, 0)
        return -dv_of(i) * acc_s[...]

    def phase_b(i, v1_s):
        rows = pl.ds(i * T2, T2)
        act = row_active(i)

        @pl.when(act > 0.5)
        def _():
            m2 = spmm_rows(i, v2_s)
            v1_s[rows, :] = dv_of(i) * (v1_s[rows, :] + 2.0 * m2)

    def relu_into(i, v1_s, dst_s):
        rows = pl.ds(i * T2, T2)
        act = row_active(i)

        @pl.when(act > 0.5)
        def _():
            dst_s[rows, :] = jnp.maximum(
                base_s[rows, :] + spmm_rows(i, v1_s), 0.0)

        @pl.when(act <= 0.5)
        def _():
            dst_s[rows, :] = jnp.maximum(base_s[rows, :], 0.0)

    def head(i):
        rows = pl.ds(i * T2, T2)
        z = (jnp.dot(h1_s[rows, :], l1w[:H, :],
                     preferred_element_type=jnp.float32)
             + jnp.dot(h2_s[rows, :], l1w[H:2 * H, :],
                       preferred_element_type=jnp.float32)
             + jnp.dot(h3_s[rows, :], l1w[2 * H:, :],
                       preferred_element_type=jnp.float32)
             + l1b[...])
        z = jnp.maximum(z, 0.0)
        z = z * (_BN_SCALE * bg[...]) + bb[...]
        logit = (jnp.dot(z, l2w[...], preferred_element_type=jnp.float32)
                 + l2b[...])
        m = jnp.max(logit, axis=1, keepdims=True)
        e = jnp.exp(logit - m)
        out_ref[rows, :] = e / jnp.sum(e, axis=1, keepdims=True)

    def loop(fn):
        jax.lax.fori_loop(0, NT2, lambda i, c: (fn(i), c)[1], 0)

    def dega0(i):
        deg_dinv(i)
        phase_a(i, x_ref[pl.ds(i * T2, T2), :], wc0, v1a_s)

    loop(dega0)
    loop(lambda i: phase_b(i, v1a_s))

    def c0a1(i):
        relu_into(i, v1a_s, h1_s)
        phase_a(i, h1_s[pl.ds(i * T2, T2), :], wc1, v1b_s)

    loop(c0a1)
    loop(lambda i: phase_b(i, v1b_s))

    def c1a2(i):
        relu_into(i, v1b_s, h2_s)
        phase_a(i, h2_s[pl.ds(i * T2, T2), :], wc2, v1a_s)

    loop(c1a2)
    loop(lambda i: phase_b(i, v1a_s))

    def c2head(i):
        relu_into(i, v1a_s, h3_s)
        head(i)

    loop(c2head)


def kernel(x, w0_0, w0_1, w0_2, w1_0, w1_1, w1_2, w2_0, w2_1, w2_2,
           lin1_w, lin1_b, bn_gamma, bn_beta, lin2_w, lin2_b):
    f32 = jnp.float32
    full = pl.BlockSpec(memory_space=pltpu.VMEM)
    wc0 = jnp.concatenate([w0_0, w0_1, w0_2], axis=1)
    wc1 = jnp.concatenate([w1_0, w1_1, w1_2], axis=1)
    wc2 = jnp.concatenate([w2_0, w2_1, w2_2], axis=1)
    out = pl.pallas_call(
        _gcn_kernel,
        in_specs=[full] * 10,
        out_specs=full,
        out_shape=jax.ShapeDtypeStruct((N, 10), f32),
        scratch_shapes=[pltpu.VMEM((N, H), f32)] * 7
                       + [pltpu.VMEM((N, 1), f32)]
                       + [pltpu.VMEM((T2, H), f32)]
                       + [pltpu.VMEM((N, 1), f32),
                          pltpu.VMEM((N, D), jnp.bfloat16),
                          pltpu.SMEM((NT2 * NT2,), f32)],
    )(x, wc0, wc1, wc2,
      lin1_w, lin1_b.reshape(1, -1), bn_gamma.reshape(1, -1),
      bn_beta.reshape(1, -1), lin2_w, lin2_b.reshape(1, -1))
    return out


# speedup vs baseline: 1.1209x; 1.0277x over previous
import jax
import jax.numpy as jnp
from jax.experimental import pallas as pl
from jax.experimental.pallas import tpu as pltpu

N = 4096
D = 512
H = 128
T2 = 512
NT2 = N // T2
THR = 0.9
SUS_THR = 0.88
_BN_SCALE = 1.0 / (1.0 + 1e-5) ** 0.5


def _gcn_kernel(x_ref,
                wc0, wc1, wc2, w0_1, w1_1, w2_1, l1w, l1b, bg, bb, l2w, l2b,
                out_ref,
                v1a_s, v1b_s, v2_s, base_s, h1_s, h2_s, h3_s, dinv_s, acc_s,
                nrm_s, xnb_s, cnt_sm):
    nrm_ref = nrm_s

    def norm_body(i, c):
        rows = pl.ds(i * T2, T2)
        xi = x_ref[rows, :]
        nrm = jnp.maximum(jnp.sqrt(jnp.sum(xi * xi, axis=1, keepdims=True)),
                          1e-12)
        nrm_s[rows, :] = nrm
        xnb_s[rows, :] = (xi / nrm).astype(jnp.bfloat16)
        return c

    jax.lax.fori_loop(0, NT2, norm_body, 0)

    def zero_body(k, c):
        cnt_sm[k] = 0.0
        return c

    jax.lax.fori_loop(0, NT2 * NT2, zero_body, 0)

    def pair_body(k, c):
        i = k // NT2
        j = k % NT2

        @pl.when(j >= i)
        def _():
            xib = xnb_s[pl.ds(i * T2, T2), :]
            xjb = xnb_s[pl.ds(j * T2, T2), :]
            s = jax.lax.dot_general(xib, xjb, (((1,), (1,)), ((), ())),
                                    preferred_element_type=jnp.float32)
            colg = jax.lax.broadcasted_iota(jnp.int32, (T2, T2), 1)
            rowg = jax.lax.broadcasted_iota(jnp.int32, (T2, T2), 0)
            off_diag = (colg > rowg) | (j != i)
            sus = ((s > SUS_THR) & off_diag).astype(jnp.float32)
            v = jnp.sum(sus)
            cnt_sm[i * NT2 + j] += v
            cnt_sm[j * NT2 + i] += v
        return c

    jax.lax.fori_loop(0, NT2 * NT2, pair_body, 0)

    def dv_of(i):
        return dinv_s[pl.ds(i * T2, T2), :]

    def row_active(i):
        def rbody(j, r):
            return jnp.maximum(r, cnt_sm[i * NT2 + j])
        return jax.lax.fori_loop(0, NT2, rbody, jnp.float32(0.0))

    def exact_tile(i, j):
        rows = pl.ds(i * T2, T2)
        cols = pl.ds(j * T2, T2)
        xi = x_ref[rows, :] / nrm_ref[rows, :]
        xj = x_ref[cols, :] / nrm_ref[cols, :]
        s = jax.lax.dot_general(xi, xj, (((1,), (1,)), ((), ())),
                                preferred_element_type=jnp.float32)
        colg = jax.lax.broadcasted_iota(jnp.int32, (T2, T2), 1) + j * T2
        rowg = jax.lax.broadcasted_iota(jnp.int32, (T2, T2), 0) + i * T2
        return jnp.where((s > THR) & (colg != rowg), s, 0.0)

    def deg_dinv(i):
        rows = pl.ds(i * T2, T2)
        act = row_active(i)

        @pl.when(act > 0.5)
        def _():
            acc_s[...] = jnp.zeros((T2, H), jnp.float32)

            def dbody(j, carry):
                @pl.when(cnt_sm[i * NT2 + j] > 0.5)
                def _():
                    a = exact_tile(i, j)
                    acc_s[:, 0:1] += jnp.sum(a, axis=1, keepdims=True)
                return carry

            jax.lax.fori_loop(0, NT2, dbody, 0)
            deg = acc_s[:, 0:1]
            dinv_s[rows, :] = jnp.where(
                deg > 0, jax.lax.rsqrt(jnp.maximum(deg, 1e-12)), 0.0)

        @pl.when(act <= 0.5)
        def _():
            dinv_s[rows, :] = jnp.zeros((T2, 1), jnp.float32)

    def phase_a(i, h, wcat, w1, v1_s):
        rows = pl.ds(i * T2, T2)
        bv = jnp.dot(h, wcat[...], preferred_element_type=jnp.float32)
        b0, v2 = bv[:, :H], bv[:, H:]
        base_s[rows, :] = b0 - v2

        @pl.when(row_active(i) > 0.5)
        def _():
            v1_s[rows, :] = jnp.dot(h, w1[...],
                                    preferred_element_type=jnp.float32)
            v2_s[rows, :] = dv_of(i) * v2

    def spmm_rows(i, src_s):
        acc_s[...] = jnp.zeros((T2, H), jnp.float32)

        def body(j, carry):
            @pl.when(cnt_sm[i * NT2 + j] > 0.5)
            def _():
                a = exact_tile(i, j)
                acc_s[...] += jnp.dot(a, src_s[pl.ds(j * T2, T2), :],
                                      preferred_element_type=jnp.float32)
            return carry

        jax.lax.fori_loop(0, NT2, body, 0)
        return -dv_of(i) * acc_s[...]

    def phase_b(i, v1_s):
        rows = pl.ds(i * T2, T2)
        act = row_active(i)

        @pl.when(act > 0.5)
        def _():
            m2 = spmm_rows(i, v2_s)
            v1_s[rows, :] = dv_of(i) * (v1_s[rows, :] + 2.0 * m2)

    def relu_into(i, v1_s, dst_s):
        rows = pl.ds(i * T2, T2)
        act = row_active(i)

        @pl.when(act > 0.5)
        def _():
            dst_s[rows, :] = jnp.maximum(
                base_s[rows, :] + spmm_rows(i, v1_s), 0.0)

        @pl.when(act <= 0.5)
        def _():
            dst_s[rows, :] = jnp.maximum(base_s[rows, :], 0.0)

    def head(i):
        rows = pl.ds(i * T2, T2)
        z = (jnp.dot(h1_s[rows, :], l1w[:H, :],
                     preferred_element_type=jnp.float32)
             + jnp.dot(h2_s[rows, :], l1w[H:2 * H, :],
                       preferred_element_type=jnp.float32)
             + jnp.dot(h3_s[rows, :], l1w[2 * H:, :],
                       preferred_element_type=jnp.float32)
             + l1b[...])
        z = jnp.maximum(z, 0.0)
        z = z * (_BN_SCALE * bg[...]) + bb[...]
        logit = (jnp.dot(z, l2w[...], preferred_element_type=jnp.float32)
                 + l2b[...])
        m = jnp.max(logit, axis=1, keepdims=True)
        e = jnp.exp(logit - m)
        out_ref[rows, :] = e / jnp.sum(e, axis=1, keepdims=True)

    def loop(fn):
        jax.lax.fori_loop(0, NT2, lambda i, c: (fn(i), c)[1], 0)

    def dega0(i):
        deg_dinv(i)
        phase_a(i, x_ref[pl.ds(i * T2, T2), :], wc0, w0_1, v1a_s)

    loop(dega0)
    loop(lambda i: phase_b(i, v1a_s))

    def c0a1(i):
        relu_into(i, v1a_s, h1_s)
        phase_a(i, h1_s[pl.ds(i * T2, T2), :], wc1, w1_1, v1b_s)

    loop(c0a1)
    loop(lambda i: phase_b(i, v1b_s))

    def c1a2(i):
        relu_into(i, v1b_s, h2_s)
        phase_a(i, h2_s[pl.ds(i * T2, T2), :], wc2, w2_1, v1a_s)

    loop(c1a2)
    loop(lambda i: phase_b(i, v1a_s))

    def c2head(i):
        relu_into(i, v1a_s, h3_s)
        head(i)

    loop(c2head)


def kernel(x, w0_0, w0_1, w0_2, w1_0, w1_1, w1_2, w2_0, w2_1, w2_2,
           lin1_w, lin1_b, bn_gamma, bn_beta, lin2_w, lin2_b):
    f32 = jnp.float32
    full = pl.BlockSpec(memory_space=pltpu.VMEM)
    wc0 = jnp.concatenate([w0_0, w0_2], axis=1)
    wc1 = jnp.concatenate([w1_0, w1_2], axis=1)
    wc2 = jnp.concatenate([w2_0, w2_2], axis=1)
    out = pl.pallas_call(
        _gcn_kernel,
        in_specs=[full] * 13,
        out_specs=full,
        out_shape=jax.ShapeDtypeStruct((N, 10), f32),
        scratch_shapes=[pltpu.VMEM((N, H), f32)] * 7
                       + [pltpu.VMEM((N, 1), f32)]
                       + [pltpu.VMEM((T2, H), f32)]
                       + [pltpu.VMEM((N, 1), f32),
                          pltpu.VMEM((N, D), jnp.bfloat16),
                          pltpu.SMEM((NT2 * NT2,), f32)],
    )(x, wc0, wc1, wc2, w0_1, w1_1, w2_1,
      lin1_w, lin1_b.reshape(1, -1), bn_gamma.reshape(1, -1),
      bn_beta.reshape(1, -1), lin2_w, lin2_b.reshape(1, -1))
    return out
